# Initial kernel scaffold; baseline (speedup 1.0000x reference)
import jax
import jax.numpy as jnp
from jax.experimental import pallas as pl


def kernel(nodes, edges, senders, receivers, copy_arr, Wq, bq, W1, b1, g1, be1, W2, b2, g2, be2, W3, b3, g3, be3):
    raise NotImplementedError("write your pallas kernel here")


# speedup vs baseline: 31.8924x; 31.8924x over previous
import functools

import jax
import jax.numpy as jnp
from jax import lax
from jax.experimental import pallas as pl
from jax.experimental.pallas import tpu as pltpu
from jax.experimental.pallas import tpu_sc as plsc

_NC = 2
_NS = 16
_NW = _NC * _NS


def _mm_body(n_ref, w_ref, b_ref, h_ref):
    h_ref[...] = (
        jnp.dot(n_ref[...], w_ref[...], preferred_element_type=jnp.float32)
        + b_ref[...]
    )


def _fin_body(p_ref, c_ref, h_ref, o_ref):
    s = p_ref[0] + p_ref[1] + h_ref[...]
    cnt = c_ref[0, :, 0:1] + c_ref[1, :, 0:1] + 1.0
    x = s / cnt
    o_ref[...] = jnp.where(x >= 0.0, x, 0.01 * x)


@functools.lru_cache(maxsize=None)
def _make_sc_agg(n, proj, ch, chunk):
    rpt = n // _NS

    def body(h_hbm, snd_hbm, rcv_hbm, z64_hbm, z16_hbm, ones_hbm,
             part_hbm, cnt_hbm,
             s_v, r_v, rows_v, ones_v, acc_sh, cnt_sh, sem):
        cid = lax.axis_index("c")
        sid = lax.axis_index("s")
        wid = sid * _NC + cid

        pltpu.sync_copy(z64_hbm, acc_sh.at[pl.ds(sid * rpt, rpt)])
        pltpu.sync_copy(z16_hbm, cnt_sh.at[pl.ds(sid * rpt, rpt)])
        pltpu.sync_copy(snd_hbm.at[wid], s_v)
        pltpu.sync_copy(rcv_hbm.at[wid], r_v)
        pltpu.sync_copy(ones_hbm, ones_v)
        plsc.subcore_barrier()

        @pl.loop(0, ch)
        def _(j):
            pltpu.async_copy(h_hbm.at[s_v.at[j]], rows_v, sem).wait()
            pltpu.sync_copy(rows_v, acc_sh.at[r_v.at[j]], add=True)
            pltpu.sync_copy(ones_v, cnt_sh.at[r_v.at[j]], add=True)

        plsc.subcore_barrier()
        pltpu.sync_copy(acc_sh.at[pl.ds(sid * rpt, rpt)],
                        part_hbm.at[cid, pl.ds(sid * rpt, rpt)])
        pltpu.sync_copy(cnt_sh.at[pl.ds(sid * rpt, rpt)],
                        cnt_hbm.at[cid, pl.ds(sid * rpt, rpt)])

    return pl.kernel(
        body,
        out_type=[
            jax.ShapeDtypeStruct((_NC, n, proj), jnp.float32),
            jax.ShapeDtypeStruct((_NC, n, 16), jnp.float32),
        ],
        mesh=plsc.VectorSubcoreMesh(core_axis_name="c", subcore_axis_name="s"),
        scratch_types=[
            pltpu.VMEM((ch, chunk), jnp.int32),
            pltpu.VMEM((ch, chunk), jnp.int32),
            pltpu.VMEM((chunk, proj), jnp.float32),
            pltpu.VMEM((chunk, 16), jnp.float32),
            pltpu.VMEM_SHARED((n, proj), jnp.float32),
            pltpu.VMEM_SHARED((n, 16), jnp.float32),
            pltpu.SemaphoreType.DMA,
        ],
        compiler_params=pltpu.CompilerParams(use_tc_tiling_on_sc=False),
    )


def kernel(nodes, edges, senders, receivers, copy_arr, Wq, bq, W1, b1, g1,
           be1, W2, b2, g2, be2, W3, b3, g3, be3):
    n, _ = nodes.shape
    proj = Wq.shape[1]
    e = senders.shape[0]
    ept = e // _NW
    chunk = 80
    ch = ept // chunk

    h = pl.pallas_call(
        _mm_body,
        out_shape=jax.ShapeDtypeStruct((n, proj), jnp.float32),
    )(nodes, Wq, bq.reshape(1, proj))

    snd = senders.reshape(_NW, ch, chunk)
    rcv = receivers.reshape(_NW, ch, chunk)
    z64 = jnp.zeros((n // _NS, proj), jnp.float32)
    z16 = jnp.zeros((n // _NS, 16), jnp.float32)
    ones = jnp.ones((chunk, 16), jnp.float32)

    part, cnt = _make_sc_agg(n, proj, ch, chunk)(h, snd, rcv, z64, z16, ones)

    out = pl.pallas_call(
        _fin_body,
        out_shape=jax.ShapeDtypeStruct((n, proj), jnp.float32),
    )(part, cnt, h)
    return out
